# baseline (speedup 1.0000x reference)
import jax
import jax.numpy as jnp
from jax.experimental import pallas as pl


def _body(x_ref, m_ref, o_ref):
    x = x_ref[...]
    mx = jnp.max(x, axis=1, keepdims=True)
    e = jnp.exp(x - mx) * m_ref[...]
    s = jnp.sum(e, axis=1, keepdims=True)
    o_ref[...] = e * (1.0 / s)


def kernel(input, mask):
    B, V = input.shape
    BLK = 8
    return pl.pallas_call(
        _body,
        grid=(B // BLK,),
        in_specs=[
            pl.BlockSpec((BLK, V), lambda i: (i, 0)),
            pl.BlockSpec((BLK, V), lambda i: (i, 0)),
        ],
        out_specs=pl.BlockSpec((BLK, V), lambda i: (i, 0)),
        out_shape=jax.ShapeDtypeStruct((B, V), input.dtype),
    )(input, mask)
